# baseline (speedup 1.0000x reference)
import functools

import jax
import jax.numpy as jnp
from jax import lax
from jax.experimental import pallas as pl
from jax.experimental.pallas import tpu as pltpu
from jax.experimental.pallas import tpu_sc as plsc

_N = 320000
_D = 128
_NC = 2
_NS = 16
_NW = _NC * _NS
_ROWS = _N // _NW
_CH = 200
_NCH = _ROWS // _CH
_NB = 4

_mesh = plsc.VectorSubcoreMesh(core_axis_name="c", subcore_axis_name="s")


@functools.partial(
    pl.kernel,
    mesh=_mesh,
    out_type=jax.ShapeDtypeStruct((_N, _D), jnp.float32),
    scratch_types=(
        [pltpu.VMEM((_CH, _D), jnp.float32) for _ in range(_NB)]
        + [pltpu.SemaphoreType.DMA for _ in range(2 * _NB)]
    ),
)
def _sc_copy(src_hbm, out_hbm, *scratch):
    bufs = scratch[:_NB]
    rsems = scratch[_NB:2 * _NB]
    wsems = scratch[2 * _NB:]
    wid = lax.axis_index("s") * _NC + lax.axis_index("c")
    base = wid * _ROWS

    def rd(j):
        off = base + j * _CH
        return pltpu.async_copy(src_hbm.at[pl.ds(off, _CH)], bufs[j % _NB],
                                rsems[j % _NB])

    def wr(j):
        off = base + j * _CH
        return pltpu.async_copy(bufs[j % _NB], out_hbm.at[pl.ds(off, _CH)],
                                wsems[j % _NB])

    rhs, whs = {}, {}
    for j in range(_NCH):
        if j >= _NB:
            whs[j - _NB].wait()
        rhs[j] = rd(j)
        k = j - (_NB - 1)
        if k >= 0:
            rhs[k].wait()
            whs[k] = wr(k)
    for k in range(max(0, _NCH - (_NB - 1)), _NCH):
        rhs[k].wait()
        whs[k] = wr(k)
    for k in range(max(0, _NCH - _NB), _NCH):
        whs[k].wait()


def kernel(ref_bxyz, ref_feat, group_ids):
    del ref_bxyz, group_ids
    return _sc_copy(ref_feat)


# speedup vs baseline: 1.0204x; 1.0204x over previous
import functools

import jax
import jax.numpy as jnp
from jax import lax
from jax.experimental import pallas as pl
from jax.experimental.pallas import tpu as pltpu
from jax.experimental.pallas import tpu_sc as plsc

_N = 320000
_D = 128
_NC = 2
_NS = 16
_NW = _NC * _NS
_ROWS = _N // _NW
_CH = 496
_NFULL = _ROWS // _CH
_REM = _ROWS - _NFULL * _CH
_NCH = _NFULL + (1 if _REM else 0)

_mesh = plsc.VectorSubcoreMesh(core_axis_name="c", subcore_axis_name="s")


def _chunk_rows(j):
    return _CH if j < _NFULL else _REM


@functools.partial(
    pl.kernel,
    mesh=_mesh,
    out_type=jax.ShapeDtypeStruct((_N, _D), jnp.float32),
    scratch_types=[
        pltpu.VMEM((_CH, _D), jnp.float32),
        pltpu.VMEM((_CH, _D), jnp.float32),
        pltpu.SemaphoreType.DMA,
        pltpu.SemaphoreType.DMA,
        pltpu.SemaphoreType.DMA,
        pltpu.SemaphoreType.DMA,
    ],
)
def _sc_copy(src_hbm, out_hbm, buf0, buf1, rs0, rs1, ws0, ws1):
    bufs = (buf0, buf1)
    rsems = (rs0, rs1)
    wsems = (ws0, ws1)
    wid = lax.axis_index("s") * _NC + lax.axis_index("c")
    base = wid * _ROWS

    def rd(j):
        off = base + j * _CH
        r = _chunk_rows(j)
        return pltpu.async_copy(src_hbm.at[pl.ds(off, r)],
                                bufs[j % 2].at[pl.ds(0, r)], rsems[j % 2])

    def wr(j):
        off = base + j * _CH
        r = _chunk_rows(j)
        return pltpu.async_copy(bufs[j % 2].at[pl.ds(0, r)],
                                out_hbm.at[pl.ds(off, r)], wsems[j % 2])

    rhs = [rd(0), None]
    whs = [None, None]
    for j in range(_NCH):
        b = j % 2
        nb = (j + 1) % 2
        if j + 1 < _NCH:
            if whs[nb] is not None:
                whs[nb].wait()
            rhs[nb] = rd(j + 1)
        rhs[b].wait()
        whs[b] = wr(j)
    for h in whs:
        if h is not None:
            h.wait()


def kernel(ref_bxyz, ref_feat, group_ids):
    del ref_bxyz, group_ids
    return _sc_copy(ref_feat)


# speedup vs baseline: 1.3304x; 1.3038x over previous
import jax
import jax.numpy as jnp
from jax.experimental import pallas as pl

_BLK = 20000


def _copy_block(feat_ref, out_ref):
    out_ref[...] = feat_ref[...]


def kernel(ref_bxyz, ref_feat, group_ids):
    del ref_bxyz, group_ids
    n, d = ref_feat.shape
    grid = n // _BLK
    return pl.pallas_call(
        _copy_block,
        grid=(grid,),
        in_specs=[pl.BlockSpec((_BLK, d), lambda i: (i, 0))],
        out_specs=pl.BlockSpec((_BLK, d), lambda i: (i, 0)),
        out_shape=jax.ShapeDtypeStruct((n, d), ref_feat.dtype),
    )(ref_feat)
